# Initial kernel scaffold; baseline (speedup 1.0000x reference)
import jax
import jax.numpy as jnp
from jax.experimental import pallas as pl


def kernel(x, params):
    raise NotImplementedError("write your pallas kernel here")


# speedup vs baseline: 68.4526x; 68.4526x over previous
import functools

import jax
import jax.numpy as jnp
from jax.experimental import pallas as pl

G = 12
RN = G * 10
TILES = 90
PB = TILES * G

_HI = jax.lax.Precision.HIGHEST


def _ln(v, g, b):
    m = jnp.mean(v, axis=1, keepdims=True)
    var = jnp.mean((v - m) ** 2, axis=1, keepdims=True)
    return (v - m) / jnp.sqrt(var + 1e-5) * g + b


def _elu(v):
    return jnp.where(v > 0, v, jnp.exp(jnp.minimum(v, 0.0)) - 1.0)


def _gat_head(hw_h, a_s, a_d, C):
    es = jax.lax.dot_general(hw_h, a_s, (((1,), (1,)), ((), ())),
                             precision=_HI)
    ed = jax.lax.dot_general(a_d, hw_h, (((1,), (1,)), ((), ())),
                             precision=_HI)
    e = es + ed
    e = jnp.where(e > 0, e, 0.2 * e)
    emax = jnp.max(jnp.where(C > 0, e, -1e30), axis=0, keepdims=True)
    emax = jnp.where(emax < -1e29, 0.0, emax)
    ee = jnp.exp(e - emax)
    den = jnp.sum(C * ee, axis=0, keepdims=True)
    w = C * ee / (den + 1e-16)
    return jax.lax.dot_general(w, hw_h, (((0,), (0,)), ((), ())),
                               precision=_HI)


def _graph_kernel(nf_ref, topo_ref,
                  we, be, gne, bne, wp, bp,
                  wg1, as1, ad1, bg1, g1, b1,
                  wg2, as2, ad2, bg2, g2, b2,
                  out_ref):
    f32 = jnp.float32
    nf = nf_ref[0]
    topo = topo_ref[0]

    h0 = jax.nn.relu(jnp.dot(nf, we[...], precision=_HI) + be[...])
    h0 = _ln(h0, gne[...], bne[...])
    ident = jnp.dot(h0, wp[...], precision=_HI) + bp[...]

    mask = (topo != 0).astype(f32)
    rowsum = jnp.sum(mask, axis=1, keepdims=True)
    gi = jax.lax.broadcasted_iota(jnp.int32, (G, RN), 0)
    ri = jax.lax.broadcasted_iota(jnp.int32, (G, RN), 1)
    seg = (ri // 10 == gi).astype(f32)
    nnz = jnp.dot(seg, rowsum, precision=_HI)
    gi2 = jax.lax.broadcasted_iota(jnp.int32, (RN, G), 1)
    ri2 = jax.lax.broadcasted_iota(jnp.int32, (RN, G), 0)
    segT = (ri2 // 10 == gi2).astype(f32)
    padrow = jnp.dot(segT, 100.0 - nnz, precision=_HI)
    r2 = jax.lax.broadcasted_iota(jnp.int32, (RN, RN), 0)
    c2 = jax.lax.broadcasted_iota(jnp.int32, (RN, RN), 1)
    sameg = (r2 // 10) == (c2 // 10)
    C = jnp.where(sameg, jnp.tile(mask, (1, G)), 0.0)
    ind0 = ((r2 % 10 == 0) & (c2 == r2)).astype(f32)
    C = C + ind0 * padrow

    hw1 = jnp.dot(h0, wg1[...], precision=_HI)
    outs = []
    for h in range(4):
        hw_h = hw1[:, 64 * h:64 * (h + 1)]
        outs.append(_gat_head(hw_h, as1[h:h + 1, :], ad1[h:h + 1, :], C))
    x1 = jnp.concatenate(outs, axis=1) + bg1[...]
    x1 = _elu(_ln(x1, g1[...], b1[...]))

    hw2 = jnp.dot(x1, wg2[...], precision=_HI)
    x2 = _gat_head(hw2, as2[...], ad2[...], C) + bg2[...]
    x2 = _ln(x2, g2[...], b2[...])

    outg = _elu(x2 + ident)
    out_ref[0] = jnp.dot(seg, outg, precision=_HI) * 0.1


def _head_kernel(g_ref, rt_ref, tf_ref,
                 wr, br, gr, brb, wt, bt, gt, btb,
                 wf, bf, gf, bfb, wa1, ba1, wa2, ba2, wa3, ba3,
                 out_ref):
    g = g_ref[...]
    r = _ln(jax.nn.relu(jnp.dot(rt_ref[...], wr[...], precision=_HI) + br[...]),
            gr[...], brb[...])
    t = _ln(jax.nn.relu(jnp.dot(tf_ref[...], wt[...], precision=_HI) + bt[...]),
            gt[...], btb[...])
    comb = jnp.concatenate([g, r, t], axis=1)
    feat = _ln(jax.nn.relu(jnp.dot(comb, wf[...], precision=_HI) + bf[...]),
               gf[...], bfb[...])
    h1 = jax.nn.relu(jnp.dot(feat, wa1[...], precision=_HI) + ba1[...])
    h2 = jax.nn.relu(jnp.dot(h1, wa2[...], precision=_HI) + ba2[...])
    out_ref[...] = jnp.dot(h2, wa3[...], precision=_HI) + ba3[...]


def _full(a):
    a = jnp.asarray(a, jnp.float32)
    if a.ndim == 1:
        a = a.reshape(1, -1)
    return pl.BlockSpec(a.shape, lambda i: (0,) * a.ndim), a


@jax.jit
def kernel(x, params):
    p = params
    B = x.shape[0]
    xp = jnp.pad(x, ((0, PB - B), (0, 0)))
    topo = xp[:, :100].reshape(TILES, RN, 10)
    nf = jnp.concatenate(
        [xp[:, 100:130].reshape(PB, 10, 3), xp[:, 245:255].reshape(PB, 10, 1)],
        axis=-1).reshape(TILES, RN, 4)

    pnames = ['we', 'be', 'gne', 'bne', 'wp', 'bp',
              'wg1', 'as1', 'ad1', 'bg1', 'g1', 'b1',
              'wg2', 'as2', 'ad2', 'bg2', 'g2', 'b2']
    specs, vals = zip(*(_full(p[n]) for n in pnames))

    g_all = pl.pallas_call(
        _graph_kernel,
        grid=(TILES,),
        in_specs=[pl.BlockSpec((1, RN, 4), lambda i: (i, 0, 0)),
                  pl.BlockSpec((1, RN, 10), lambda i: (i, 0, 0)),
                  *specs],
        out_specs=pl.BlockSpec((1, G, 64), lambda i: (i, 0, 0)),
        out_shape=jax.ShapeDtypeStruct((TILES, G, 64), jnp.float32),
    )(nf, topo, *vals)
    g_all = g_all.reshape(PB, 64)[:B]

    routing = x[:, 130:140]
    traffic = x[:, 240:245]
    hnames = ['wr', 'br', 'gr', 'brb', 'wt', 'bt', 'gt', 'btb',
              'wf', 'bf', 'gf', 'bfb', 'wa1', 'ba1', 'wa2', 'ba2',
              'wa3', 'ba3']
    hspecs, hvals = zip(*(_full(p[n]) for n in hnames))
    BB = 256
    out = pl.pallas_call(
        _head_kernel,
        grid=(B // BB,),
        in_specs=[pl.BlockSpec((BB, 64), lambda i: (i, 0)),
                  pl.BlockSpec((BB, 10), lambda i: (i, 0)),
                  pl.BlockSpec((BB, 5), lambda i: (i, 0)),
                  *hspecs],
        out_specs=pl.BlockSpec((BB, 10), lambda i: (i, 0)),
        out_shape=jax.ShapeDtypeStruct((B, 10), jnp.float32),
    )(g_all, routing, traffic, *hvals)
    return out


# speedup vs baseline: 79.5292x; 1.1618x over previous
import jax
import jax.numpy as jnp
from jax.experimental import pallas as pl

G = 12
RN = G * 10
SUB = 2
SPG = G * SUB
ROWS = RN * SUB
STEPS = 45
PB = STEPS * SPG

_HI = jax.lax.Precision.HIGHEST


def _ln(v, g, b):
    m = jnp.mean(v, axis=1, keepdims=True)
    var = jnp.mean((v - m) ** 2, axis=1, keepdims=True)
    return (v - m) / jnp.sqrt(var + 1e-5) * g + b


def _elu(v):
    return jnp.where(v > 0, v, jnp.exp(jnp.minimum(v, 0.0)) - 1.0)


def _gat_head(hw_h, a_s, a_d, C, neg):
    es = jax.lax.dot_general(hw_h, a_s, (((1,), (1,)), ((), ())),
                             precision=_HI)
    ed = jax.lax.dot_general(a_d, hw_h, (((1,), (1,)), ((), ())),
                             precision=_HI)
    e = es + ed
    e = jnp.maximum(e, 0.2 * e)
    emax = jnp.max(e + neg, axis=0, keepdims=True)
    emax = jnp.where(emax < -1e29, 0.0, emax)
    cee = C * jnp.exp(e - emax)
    den = jnp.sum(cee, axis=0, keepdims=True)
    w = cee * (1.0 / (den + 1e-16))
    return jax.lax.dot_general(w, hw_h, (((0,), (0,)), ((), ())),
                               precision=_HI)


def _actor_kernel(nf_ref, topo_ref, rt_ref, tf_ref,
                  segf, segft, samegf, ind0f, rep,
                  we, be, gne, bne, wp, bp,
                  wg1, as1, ad1, bg1, g1, b1,
                  wg2, as2, ad2, bg2, g2, b2,
                  wr, br, gr, brb, wt, bt, gt, btb,
                  wf, bf, gf, bfb, wa1, ba1, wa2, ba2, wa3, ba3,
                  out_ref):
    nf = nf_ref[0]
    topo = topo_ref[0]

    h0 = jax.nn.relu(jnp.dot(nf, we[...], precision=_HI) + be[...])
    h0 = _ln(h0, gne[...], bne[...])
    ident = jnp.dot(h0, wp[...], precision=_HI) + bp[...]
    hw1 = jnp.dot(h0, wg1[...], precision=_HI)

    mask = (topo != 0).astype(jnp.float32)
    rowsum = jnp.sum(mask, axis=1, keepdims=True)
    nnz = jnp.dot(segf[...], rowsum, precision=_HI)
    padrow = jnp.dot(segft[...], 100.0 - nnz,
                     precision=_HI)
    tilec = jnp.dot(mask, rep[...], precision=_HI)

    Cs, negs = [], []
    for t in range(SUB):
        sl = slice(RN * t, RN * (t + 1))
        C = tilec[sl] * samegf[...] + ind0f[...] * padrow[sl]
        Cs.append(C)
        negs.append(jnp.where(C > 0, 0.0, -1e30))

    x1_parts = []
    for t in range(SUB):
        sl = slice(RN * t, RN * (t + 1))
        outs = [_gat_head(hw1[sl, 64 * h:64 * (h + 1)],
                          as1[h:h + 1, :], ad1[h:h + 1, :], Cs[t], negs[t])
                for h in range(4)]
        x1_parts.append(jnp.concatenate(outs, axis=1))
    x1 = jnp.concatenate(x1_parts, axis=0) + bg1[...]
    x1 = _elu(_ln(x1, g1[...], b1[...]))

    hw2 = jnp.dot(x1, wg2[...], precision=_HI)
    x2 = jnp.concatenate(
        [_gat_head(hw2[RN * t:RN * (t + 1)], as2[...], ad2[...],
                   Cs[t], negs[t]) for t in range(SUB)],
        axis=0) + bg2[...]
    x2 = _ln(x2, g2[...], b2[...])

    outg = _elu(x2 + ident)
    g = jnp.dot(segf[...], outg, precision=_HI) * 0.1

    r = _ln(jax.nn.relu(jnp.dot(rt_ref[0], wr[...], precision=_HI) + br[...]),
            gr[...], brb[...])
    tt = _ln(jax.nn.relu(jnp.dot(tf_ref[0], wt[...], precision=_HI) + bt[...]),
             gt[...], btb[...])
    comb = jnp.concatenate([g, r, tt], axis=1)
    feat = _ln(jax.nn.relu(jnp.dot(comb, wf[...], precision=_HI) + bf[...]),
               gf[...], bfb[...])
    h1 = jax.nn.relu(jnp.dot(feat, wa1[...], precision=_HI) + ba1[...])
    h2 = jax.nn.relu(jnp.dot(h1, wa2[...], precision=_HI) + ba2[...])
    out_ref[0] = jnp.dot(h2, wa3[...], precision=_HI) + ba3[...]


def _full(a):
    a = jnp.asarray(a, jnp.float32)
    if a.ndim == 1:
        a = a.reshape(1, -1)
    return pl.BlockSpec(a.shape, lambda i: (0,) * a.ndim), a


@jax.jit
def kernel(x, params):
    p = params
    B = x.shape[0]
    xp = jnp.pad(x, ((0, PB - B), (0, 0)))
    topo = xp[:, :100].reshape(STEPS, ROWS, 10)
    nf = jnp.concatenate(
        [xp[:, 100:130].reshape(PB, 10, 3), xp[:, 245:255].reshape(PB, 10, 1)],
        axis=-1).reshape(STEPS, ROWS, 4)
    routing = xp[:, 130:140].reshape(STEPS, SPG, 10)
    traffic = xp[:, 240:245].reshape(STEPS, SPG, 5)

    s_of_row = jnp.arange(ROWS) // 10
    segf = (s_of_row[None, :] == jnp.arange(SPG)[:, None]).astype(jnp.float32)
    segft = segf.T
    r1 = jnp.arange(RN)
    samegf = ((r1[:, None] // 10) == (r1[None, :] // 10)).astype(jnp.float32)
    ind0f = ((r1[:, None] % 10 == 0) & (r1[None, :] == r1[:, None])
             ).astype(jnp.float32)
    rep = (jnp.arange(10)[:, None] == (r1[None, :] % 10)).astype(jnp.float32)

    pnames = ['we', 'be', 'gne', 'bne', 'wp', 'bp',
              'wg1', 'as1', 'ad1', 'bg1', 'g1', 'b1',
              'wg2', 'as2', 'ad2', 'bg2', 'g2', 'b2',
              'wr', 'br', 'gr', 'brb', 'wt', 'bt', 'gt', 'btb',
              'wf', 'bf', 'gf', 'bfb', 'wa1', 'ba1', 'wa2', 'ba2',
              'wa3', 'ba3']
    cspecs, cvals = zip(*(_full(a) for a in
                          (segf, segft, samegf, ind0f, rep)))
    pspecs, pvals = zip(*(_full(p[n]) for n in pnames))

    out = pl.pallas_call(
        _actor_kernel,
        grid=(STEPS,),
        in_specs=[pl.BlockSpec((1, ROWS, 4), lambda i: (i, 0, 0)),
                  pl.BlockSpec((1, ROWS, 10), lambda i: (i, 0, 0)),
                  pl.BlockSpec((1, SPG, 10), lambda i: (i, 0, 0)),
                  pl.BlockSpec((1, SPG, 5), lambda i: (i, 0, 0)),
                  *cspecs, *pspecs],
        out_specs=pl.BlockSpec((1, SPG, 10), lambda i: (i, 0, 0)),
        out_shape=jax.ShapeDtypeStruct((STEPS, SPG, 10), jnp.float32),
    )(nf, topo, routing, traffic, *cvals, *pvals)
    return out.reshape(PB, 10)[:B]


# speedup vs baseline: 88.7980x; 1.1165x over previous
import jax
import jax.numpy as jnp
from jax.experimental import pallas as pl

G = 12
RN = G * 10
SUB = 4
SPG = G * SUB
ROWS = RN * SUB
STEPS = 23
PB = STEPS * SPG

_HI = jax.lax.Precision.HIGHEST


def _ln(v, g, b):
    m = jnp.mean(v, axis=1, keepdims=True)
    var = jnp.mean((v - m) ** 2, axis=1, keepdims=True)
    return (v - m) / jnp.sqrt(var + 1e-5) * g + b


def _elu(v):
    return jnp.where(v > 0, v, jnp.exp(jnp.minimum(v, 0.0)) - 1.0)


def _gat_head(hw_h, a_s, a_d, CT, neg):
    esr = jax.lax.dot_general(a_s, hw_h, (((1,), (1,)), ((), ())),
                              precision=_HI)
    edc = jax.lax.dot_general(hw_h, a_d, (((1,), (1,)), ((), ())),
                              precision=_HI)
    e = edc + esr
    e = jnp.maximum(e, 0.2 * e)
    emax = jnp.max(e + neg, axis=1, keepdims=True)
    emax = jnp.where(emax < -1e29, 0.0, emax)
    cee = CT * jnp.exp(e - emax)
    den = jnp.sum(cee, axis=1, keepdims=True)
    w = cee * (1.0 / (den + 1e-16))
    return jnp.dot(w, hw_h, precision=_HI)


def _actor_kernel(nf_ref, topot_ref, rt_ref, tf_ref,
                  segf, segft, samegf, ind0f, rept,
                  we, be, gne, bne, wp, bp,
                  wg1, as1, ad1, bg1, g1, b1,
                  wg2, as2, ad2, bg2, g2, b2,
                  wr, br, gr, brb, wt, bt, gt, btb,
                  wf, bf, gf, bfb, wa1, ba1, wa2, ba2, wa3, ba3,
                  out_ref):
    nf = nf_ref[0]
    topot = topot_ref[0]

    h0 = jax.nn.relu(jnp.dot(nf, we[...], precision=_HI) + be[...])
    h0 = _ln(h0, gne[...], bne[...])
    ident = jnp.dot(h0, wp[...], precision=_HI) + bp[...]
    hw1 = jnp.dot(h0, wg1[...], precision=_HI)

    maskt = (topot != 0).astype(jnp.float32)
    colsum = jnp.sum(maskt, axis=0, keepdims=True)
    nnzt = jnp.dot(colsum, segft[...], precision=_HI)
    padc = jnp.dot(100.0 - nnzt, segf[...],
                   precision=_HI)

    CTs, negs = [], []
    for t in range(SUB):
        sl = slice(RN * t, RN * (t + 1))
        CT = (jnp.dot(rept[...], maskt[:, sl], precision=_HI) * samegf[...]
              + ind0f[...] * padc[:, sl])
        CTs.append(CT)
        negs.append(jnp.where(CT > 0, 0.0, -1e30))

    x1_parts = []
    for t in range(SUB):
        sl = slice(RN * t, RN * (t + 1))
        outs = [_gat_head(hw1[sl, 64 * h:64 * (h + 1)],
                          as1[h:h + 1, :], ad1[h:h + 1, :], CTs[t], negs[t])
                for h in range(4)]
        x1_parts.append(jnp.concatenate(outs, axis=1))
    x1 = jnp.concatenate(x1_parts, axis=0) + bg1[...]
    x1 = _elu(_ln(x1, g1[...], b1[...]))

    hw2 = jnp.dot(x1, wg2[...], precision=_HI)
    x2 = jnp.concatenate(
        [_gat_head(hw2[RN * t:RN * (t + 1)], as2[...], ad2[...],
                   CTs[t], negs[t]) for t in range(SUB)],
        axis=0) + bg2[...]
    x2 = _ln(x2, g2[...], b2[...])

    outg = _elu(x2 + ident)
    g = jnp.dot(segf[...], outg, precision=_HI) * 0.1

    r = _ln(jax.nn.relu(jnp.dot(rt_ref[0], wr[...], precision=_HI) + br[...]),
            gr[...], brb[...])
    tt = _ln(jax.nn.relu(jnp.dot(tf_ref[0], wt[...], precision=_HI) + bt[...]),
             gt[...], btb[...])
    comb = jnp.concatenate([g, r, tt], axis=1)
    feat = _ln(jax.nn.relu(jnp.dot(comb, wf[...], precision=_HI) + bf[...]),
               gf[...], bfb[...])
    h1 = jax.nn.relu(jnp.dot(feat, wa1[...], precision=_HI) + ba1[...])
    h2 = jax.nn.relu(jnp.dot(h1, wa2[...], precision=_HI) + ba2[...])
    out_ref[0] = jnp.dot(h2, wa3[...], precision=_HI) + ba3[...]


def _full(a):
    a = jnp.asarray(a, jnp.float32)
    if a.ndim == 1:
        a = a.reshape(1, -1)
    return pl.BlockSpec(a.shape, lambda i: (0,) * a.ndim), a


@jax.jit
def kernel(x, params):
    p = params
    B = x.shape[0]
    xp = jnp.pad(x, ((0, PB - B), (0, 0)))
    topot = (xp[:, :100].reshape(STEPS, SPG, 10, 10)
             .transpose(0, 3, 1, 2).reshape(STEPS, 10, ROWS))
    nf = jnp.concatenate(
        [xp[:, 100:130].reshape(PB, 10, 3), xp[:, 245:255].reshape(PB, 10, 1)],
        axis=-1).reshape(STEPS, ROWS, 4)
    routing = xp[:, 130:140].reshape(STEPS, SPG, 10)
    traffic = xp[:, 240:245].reshape(STEPS, SPG, 5)

    s_of_row = jnp.arange(ROWS) // 10
    segf = (s_of_row[None, :] == jnp.arange(SPG)[:, None]).astype(jnp.float32)
    segft = segf.T
    r1 = jnp.arange(RN)
    samegf = ((r1[:, None] // 10) == (r1[None, :] // 10)).astype(jnp.float32)
    ind0f = ((r1[:, None] % 10 == 0) & (r1[None, :] == r1[:, None])
             ).astype(jnp.float32)
    rept = ((r1[:, None] % 10) == jnp.arange(10)[None, :]).astype(jnp.float32)

    pnames = ['we', 'be', 'gne', 'bne', 'wp', 'bp',
              'wg1', 'as1', 'ad1', 'bg1', 'g1', 'b1',
              'wg2', 'as2', 'ad2', 'bg2', 'g2', 'b2',
              'wr', 'br', 'gr', 'brb', 'wt', 'bt', 'gt', 'btb',
              'wf', 'bf', 'gf', 'bfb', 'wa1', 'ba1', 'wa2', 'ba2',
              'wa3', 'ba3']
    cspecs, cvals = zip(*(_full(a) for a in
                          (segf, segft, samegf, ind0f, rept)))
    pspecs, pvals = zip(*(_full(p[n]) for n in pnames))

    out = pl.pallas_call(
        _actor_kernel,
        grid=(STEPS,),
        in_specs=[pl.BlockSpec((1, ROWS, 4), lambda i: (i, 0, 0)),
                  pl.BlockSpec((1, 10, ROWS), lambda i: (i, 0, 0)),
                  pl.BlockSpec((1, SPG, 10), lambda i: (i, 0, 0)),
                  pl.BlockSpec((1, SPG, 5), lambda i: (i, 0, 0)),
                  *cspecs, *pspecs],
        out_specs=pl.BlockSpec((1, SPG, 10), lambda i: (i, 0, 0)),
        out_shape=jax.ShapeDtypeStruct((STEPS, SPG, 10), jnp.float32),
    )(nf, topot, routing, traffic, *cvals, *pvals)
    return out.reshape(PB, 10)[:B]


# speedup vs baseline: 144.9322x; 1.6322x over previous
import jax
import jax.numpy as jnp
from jax.experimental import pallas as pl

G = 12
RN = G * 10
SUB = 4
SPG = G * SUB
ROWS = RN * SUB
STEPS = 23
PB = STEPS * SPG

_HI = jax.lax.Precision.DEFAULT


def _ln(v, g, b):
    m = jnp.mean(v, axis=1, keepdims=True)
    var = jnp.mean((v - m) ** 2, axis=1, keepdims=True)
    return (v - m) / jnp.sqrt(var + 1e-5) * g + b


def _elu(v):
    return jnp.where(v > 0, v, jnp.exp(jnp.minimum(v, 0.0)) - 1.0)


def _gat_head(hw_h, a_s, a_d, CT, neg):
    esr = jax.lax.dot_general(a_s, hw_h, (((1,), (1,)), ((), ())),
                              precision=_HI)
    edc = jax.lax.dot_general(hw_h, a_d, (((1,), (1,)), ((), ())),
                              precision=_HI)
    e = edc + esr
    e = jnp.maximum(e, 0.2 * e)
    emax = jnp.max(e + neg, axis=1, keepdims=True)
    emax = jnp.where(emax < -1e29, 0.0, emax)
    cee = CT * jnp.exp(e - emax)
    den = jnp.sum(cee, axis=1, keepdims=True)
    w = cee * (1.0 / (den + 1e-16))
    return jnp.dot(w, hw_h, precision=_HI)


def _actor_kernel(nf_ref, topot_ref, rt_ref, tf_ref,
                  segf, segft, samegf, ind0f, rept,
                  we, be, gne, bne, wp, bp,
                  wg1, as1, ad1, bg1, g1, b1,
                  wg2, as2, ad2, bg2, g2, b2,
                  wr, br, gr, brb, wt, bt, gt, btb,
                  wf, bf, gf, bfb, wa1, ba1, wa2, ba2, wa3, ba3,
                  out_ref):
    nf = nf_ref[0]
    topot = topot_ref[0]

    h0 = jax.nn.relu(jnp.dot(nf, we[...], precision=_HI) + be[...])
    h0 = _ln(h0, gne[...], bne[...])
    ident = jnp.dot(h0, wp[...], precision=_HI) + bp[...]
    hw1 = jnp.dot(h0, wg1[...], precision=_HI)

    maskt = (topot != 0).astype(jnp.float32)
    colsum = jnp.sum(maskt, axis=0, keepdims=True)
    nnzt = jnp.dot(colsum, segft[...], precision=_HI)
    padc = jnp.dot(100.0 - nnzt, segf[...],
                   precision=_HI)

    CTs, negs = [], []
    for t in range(SUB):
        sl = slice(RN * t, RN * (t + 1))
        CT = (jnp.dot(rept[...], maskt[:, sl], precision=_HI) * samegf[...]
              + ind0f[...] * padc[:, sl])
        CTs.append(CT)
        negs.append(jnp.where(CT > 0, 0.0, -1e30))

    x1_parts = []
    for t in range(SUB):
        sl = slice(RN * t, RN * (t + 1))
        outs = [_gat_head(hw1[sl, 64 * h:64 * (h + 1)],
                          as1[h:h + 1, :], ad1[h:h + 1, :], CTs[t], negs[t])
                for h in range(4)]
        x1_parts.append(jnp.concatenate(outs, axis=1))
    x1 = jnp.concatenate(x1_parts, axis=0) + bg1[...]
    x1 = _elu(_ln(x1, g1[...], b1[...]))

    hw2 = jnp.dot(x1, wg2[...], precision=_HI)
    x2 = jnp.concatenate(
        [_gat_head(hw2[RN * t:RN * (t + 1)], as2[...], ad2[...],
                   CTs[t], negs[t]) for t in range(SUB)],
        axis=0) + bg2[...]
    x2 = _ln(x2, g2[...], b2[...])

    outg = _elu(x2 + ident)
    g = jnp.dot(segf[...], outg, precision=_HI) * 0.1

    r = _ln(jax.nn.relu(jnp.dot(rt_ref[0], wr[...], precision=_HI) + br[...]),
            gr[...], brb[...])
    tt = _ln(jax.nn.relu(jnp.dot(tf_ref[0], wt[...], precision=_HI) + bt[...]),
             gt[...], btb[...])
    comb = jnp.concatenate([g, r, tt], axis=1)
    feat = _ln(jax.nn.relu(jnp.dot(comb, wf[...], precision=_HI) + bf[...]),
               gf[...], bfb[...])
    h1 = jax.nn.relu(jnp.dot(feat, wa1[...], precision=_HI) + ba1[...])
    h2 = jax.nn.relu(jnp.dot(h1, wa2[...], precision=_HI) + ba2[...])
    out_ref[0] = jnp.dot(h2, wa3[...], precision=_HI) + ba3[...]


def _full(a):
    a = jnp.asarray(a, jnp.float32)
    if a.ndim == 1:
        a = a.reshape(1, -1)
    return pl.BlockSpec(a.shape, lambda i: (0,) * a.ndim), a


@jax.jit
def kernel(x, params):
    p = params
    B = x.shape[0]
    xp = jnp.pad(x, ((0, PB - B), (0, 0)))
    topot = (xp[:, :100].reshape(STEPS, SPG, 10, 10)
             .transpose(0, 3, 1, 2).reshape(STEPS, 10, ROWS))
    nf = jnp.concatenate(
        [xp[:, 100:130].reshape(PB, 10, 3), xp[:, 245:255].reshape(PB, 10, 1)],
        axis=-1).reshape(STEPS, ROWS, 4)
    routing = xp[:, 130:140].reshape(STEPS, SPG, 10)
    traffic = xp[:, 240:245].reshape(STEPS, SPG, 5)

    s_of_row = jnp.arange(ROWS) // 10
    segf = (s_of_row[None, :] == jnp.arange(SPG)[:, None]).astype(jnp.float32)
    segft = segf.T
    r1 = jnp.arange(RN)
    samegf = ((r1[:, None] // 10) == (r1[None, :] // 10)).astype(jnp.float32)
    ind0f = ((r1[:, None] % 10 == 0) & (r1[None, :] == r1[:, None])
             ).astype(jnp.float32)
    rept = ((r1[:, None] % 10) == jnp.arange(10)[None, :]).astype(jnp.float32)

    pnames = ['we', 'be', 'gne', 'bne', 'wp', 'bp',
              'wg1', 'as1', 'ad1', 'bg1', 'g1', 'b1',
              'wg2', 'as2', 'ad2', 'bg2', 'g2', 'b2',
              'wr', 'br', 'gr', 'brb', 'wt', 'bt', 'gt', 'btb',
              'wf', 'bf', 'gf', 'bfb', 'wa1', 'ba1', 'wa2', 'ba2',
              'wa3', 'ba3']
    cspecs, cvals = zip(*(_full(a) for a in
                          (segf, segft, samegf, ind0f, rept)))
    pspecs, pvals = zip(*(_full(p[n]) for n in pnames))

    out = pl.pallas_call(
        _actor_kernel,
        grid=(STEPS,),
        in_specs=[pl.BlockSpec((1, ROWS, 4), lambda i: (i, 0, 0)),
                  pl.BlockSpec((1, 10, ROWS), lambda i: (i, 0, 0)),
                  pl.BlockSpec((1, SPG, 10), lambda i: (i, 0, 0)),
                  pl.BlockSpec((1, SPG, 5), lambda i: (i, 0, 0)),
                  *cspecs, *pspecs],
        out_specs=pl.BlockSpec((1, SPG, 10), lambda i: (i, 0, 0)),
        out_shape=jax.ShapeDtypeStruct((STEPS, SPG, 10), jnp.float32),
    )(nf, topot, routing, traffic, *cvals, *pvals)
    return out.reshape(PB, 10)[:B]
